# Initial kernel scaffold; baseline (speedup 1.0000x reference)
import jax
import jax.numpy as jnp
from jax.experimental import pallas as pl


def kernel(loss):
    raise NotImplementedError("write your pallas kernel here")


# speedup vs baseline: 7.0293x; 7.0293x over previous
import functools

import jax
import jax.numpy as jnp
from jax import lax
from jax.experimental import pallas as pl
from jax.experimental.pallas import tpu as pltpu
from jax.experimental.pallas import tpu_sc as plsc

L = 16
PASSES = 30

_DNUMS = lax.GatherDimensionNumbers(
    offset_dims=(), collapsed_slice_dims=(0,), start_index_map=(0,))


def _shuffle(v, idx):
    return lax.gather(v, idx[:, None], _DNUMS, (1,),
                      mode=lax.GatherScatterMode.PROMISE_IN_BOUNDS)


def _butterfly(v, op):
    lane = lax.iota(jnp.int32, L)
    for s in (8, 4, 2, 1):
        v = op(v, _shuffle(v, jnp.bitwise_xor(lane, s)))
    return v


def _row_topk_sum(buf, n, k, unroll=8):
    nv = n // L

    def mm_body(i, carry):
        vmin, vmax = carry
        v = buf[pl.ds(i * L, L)]
        return jnp.minimum(vmin, v), jnp.maximum(vmax, v)

    vmin, vmax = lax.fori_loop(
        0, nv, mm_body,
        (jnp.full((L,), jnp.inf, jnp.float32),
         jnp.full((L,), -jnp.inf, jnp.float32)),
        unroll=unroll)
    lo = _butterfly(vmin, jnp.minimum)[0]
    hi = _butterfly(vmax, jnp.maximum)[0]

    kf = jnp.float32(k)

    def pass_body(_, carry):
        lo, hi = carry
        mid = 0.5 * lo + 0.5 * hi

        def cbody(i, cnt):
            v = buf[pl.ds(i * L, L)]
            return cnt + jnp.where(v > mid, 1.0, 0.0)

        cnt_v = lax.fori_loop(0, nv, cbody, jnp.zeros((L,), jnp.float32),
                              unroll=unroll)
        c = _butterfly(cnt_v, jnp.add)[0]
        ge = c >= kf
        return jnp.where(ge, mid, lo), jnp.where(ge, hi, mid)

    lo, hi = lax.fori_loop(0, PASSES, pass_body, (lo, hi))

    def fbody(i, carry):
        cnt, sv = carry
        v = buf[pl.ds(i * L, L)]
        m = v > hi
        return cnt + jnp.where(m, 1.0, 0.0), sv + jnp.where(m, v, 0.0)

    cnt_v, sum_v = lax.fori_loop(
        0, nv, fbody, (jnp.zeros((L,), jnp.float32), jnp.zeros((L,), jnp.float32)),
        unroll=unroll)
    c = _butterfly(cnt_v, jnp.add)[0]
    s = _butterfly(sum_v, jnp.add)[0]
    return s + (kf - c) * hi


def _sc_kernel(rows, cols, k, rows_per_w):
    nc = 2
    mesh = plsc.VectorSubcoreMesh(core_axis_name="c", subcore_axis_name="s")

    @functools.partial(
        pl.kernel,
        out_type=jax.ShapeDtypeStruct((rows // rows_per_w, L), jnp.float32),
        mesh=mesh,
        scratch_types=[
            pltpu.VMEM((cols,), jnp.float32),
            pltpu.VMEM((L,), jnp.float32),
        ],
    )
    def run(loss_hbm, out_hbm, buf, out_v):
        wid = lax.axis_index("s") * nc + lax.axis_index("c")
        lane = lax.iota(jnp.int32, L)
        total = jnp.float32(0.0)
        for r in range(rows_per_w):
            row = wid * rows_per_w + r
            pltpu.sync_copy(loss_hbm.at[row], buf)
            total = total + _row_topk_sum(buf, cols, k)
        out_v[...] = jnp.where(lane == 0, total, 0.0)
        pltpu.sync_copy(out_v, out_hbm.at[wid])

    return run


def kernel(loss):
    b = loss.shape[0]
    loss2 = loss.reshape(b, -1)
    p = loss2.shape[1]
    k = int(0.25 * p)
    nw = 32
    rows_per_w = b // nw
    partials = _sc_kernel(b, p, k, rows_per_w)(loss2)
    return jnp.sum(partials) / jnp.float32(b * k)
